# baseline (speedup 1.0000x reference)
import functools

import jax
import jax.numpy as jnp
from jax.experimental import pallas as pl
from jax.experimental.pallas import tpu as pltpu

_USER = 4096
_N = 8192
_D = 16
_BM = 512
_NB = _N // _BM
_NBU = _USER // _BM


def _proj_body(uf_ref, u1w_ref, usf_ref, u2w_ref, itf_ref, iw_ref, out_ref):
    ue1 = jnp.dot(uf_ref[...], u1w_ref[...], preferred_element_type=jnp.float32)
    ue2 = jnp.dot(usf_ref[...], u2w_ref[...], preferred_element_type=jnp.float32)
    ie = jnp.dot(itf_ref[...], iw_ref[...], preferred_element_type=jnp.float32)
    ue = jnp.concatenate([ue1, ue2], axis=1)
    out_ref[...] = jnp.concatenate([ue, ie], axis=0).astype(jnp.bfloat16)


def _layer_body(a_ref, x_ref, w_ref, alpha_ref, emb_ref, ego_ref, abf_ref,
                *, cast_a, emit_ego):
    a = a_ref[...]
    if cast_a:
        a = a.astype(jnp.bfloat16)
        abf_ref[...] = a
    acc = jnp.dot(a, x_ref[...], preferred_element_type=jnp.float32)
    alpha = alpha_ref[0, 0]
    emb = jnp.where(acc >= 0, acc, alpha * acc)
    emb_ref[...] = emb
    if emit_ego:
        ego = jnp.dot(emb, w_ref[0], preferred_element_type=jnp.float32)
        ego_ref[...] = ego.astype(jnp.bfloat16)


def _row_spec(i):
    return (i, 0)


def _const_spec(i):
    return (0, 0)


def _w_spec(i):
    return (jnp.where(i < _NBU, 0, 1), 0, 0)


def _layer_call(a, x, w_stack, alpha, *, cast_a, emit_ego):
    in_specs = [
        pl.BlockSpec((_BM, _N), _row_spec),
        pl.BlockSpec((_N, _D), _const_spec),
    ]
    operands = [a, x]
    if emit_ego:
        in_specs.append(pl.BlockSpec((1, _D, _D), _w_spec))
        operands.append(w_stack)
    else:
        in_specs.append(pl.BlockSpec((1, _D, _D), lambda i: (0, 0, 0)))
        operands.append(jnp.zeros((1, _D, _D), jnp.float32))
    in_specs.append(pl.BlockSpec((1, 1), _const_spec))
    operands.append(alpha)

    out_shape = [jax.ShapeDtypeStruct((_N, _D), jnp.float32)]
    out_specs = [pl.BlockSpec((_BM, _D), _row_spec)]
    if emit_ego:
        out_shape.append(jax.ShapeDtypeStruct((_N, _D), jnp.bfloat16))
        out_specs.append(pl.BlockSpec((_BM, _D), _row_spec))
    if cast_a:
        out_shape.append(jax.ShapeDtypeStruct((_N, _N), jnp.bfloat16))
        out_specs.append(pl.BlockSpec((_BM, _N), _row_spec))

    def body(a_ref, x_ref, w_ref, alpha_ref, *outs):
        emb_ref = outs[0]
        ego_ref = outs[1] if emit_ego else None
        abf_ref = outs[-1] if cast_a else None
        _layer_body(a_ref, x_ref, w_ref, alpha_ref, emb_ref, ego_ref, abf_ref,
                    cast_a=cast_a, emit_ego=emit_ego)

    return pl.pallas_call(
        body,
        grid=(_NB,),
        in_specs=in_specs,
        out_specs=out_specs,
        out_shape=out_shape,
        compiler_params=pltpu.CompilerParams(
            dimension_semantics=("parallel",)),
    )(*operands)


def kernel(user_social_feat, user_feat, item_feat, raitng_adj,
           user1_w, user2_w, item_w, user_w1, item_w1, user_w2, item_w2,
           prelu_a):
    ego0 = pl.pallas_call(
        _proj_body,
        out_shape=jax.ShapeDtypeStruct((_N, _D), jnp.bfloat16),
    )(user_feat, user1_w, user_social_feat, user2_w, item_feat, item_w)

    alpha = jnp.reshape(prelu_a, (1, 1))
    w1 = jnp.stack([user_w1, item_w1])
    w2 = jnp.stack([user_w2, item_w2])

    emb0, ego1 = _layer_call(raitng_adj, ego0, w1, alpha,
                             cast_a=False, emit_ego=True)
    a_bf = raitng_adj
    emb1, ego2 = _layer_call(a_bf, ego1, w2, alpha,
                             cast_a=False, emit_ego=True)
    (emb2,) = _layer_call(a_bf, ego2, None, alpha,
                          cast_a=False, emit_ego=False)

    user_embedding = jnp.concatenate(
        [emb0[:_USER], emb1[:_USER], emb2[:_USER]], axis=1)
    item_embedding = jnp.concatenate(
        [emb0[_USER:], emb1[_USER:], emb2[_USER:]], axis=1)
    return (user_embedding, item_embedding)


# speedup vs baseline: 1.0810x; 1.0810x over previous
import functools

import jax
import jax.numpy as jnp
from jax.experimental import pallas as pl
from jax.experimental.pallas import tpu as pltpu

_USER = 4096
_N = 8192
_D = 16
_BM = 512
_NB = _N // _BM
_NBU = _USER // _BM


def _proj_body(uf_ref, u1w_ref, usf_ref, u2w_ref, itf_ref, iw_ref, out_ref):
    ue1 = jnp.dot(uf_ref[...], u1w_ref[...], preferred_element_type=jnp.float32)
    ue2 = jnp.dot(usf_ref[...], u2w_ref[...], preferred_element_type=jnp.float32)
    ie = jnp.dot(itf_ref[...], iw_ref[...], preferred_element_type=jnp.float32)
    ue = jnp.concatenate([ue1, ue2], axis=1)
    out_ref[...] = jnp.concatenate([ue, ie], axis=0).astype(jnp.bfloat16)


def _layer_body(a_ref, x_ref, w_ref, alpha_ref, emb_ref, ego_ref, abf_ref,
                *, cast_a, emit_ego):
    a = a_ref[...]
    if cast_a:
        a = a.astype(jnp.bfloat16)
        abf_ref[...] = a
    acc = jnp.dot(a, x_ref[...], preferred_element_type=jnp.float32)
    alpha = alpha_ref[0, 0]
    emb = jnp.where(acc >= 0, acc, alpha * acc)
    emb_ref[...] = emb
    if emit_ego:
        ego = jnp.dot(emb, w_ref[0], preferred_element_type=jnp.float32)
        ego_ref[...] = ego.astype(jnp.bfloat16)


def _row_spec(i):
    return (i, 0)


def _const_spec(i):
    return (0, 0)


def _w_spec(i):
    return (jnp.where(i < _NBU, 0, 1), 0, 0)


def _layer_call(a, x, w_stack, alpha, *, cast_a, emit_ego):
    in_specs = [
        pl.BlockSpec((_BM, _N), _row_spec),
        pl.BlockSpec((_N, _D), _const_spec),
    ]
    operands = [a, x]
    if emit_ego:
        in_specs.append(pl.BlockSpec((1, _D, _D), _w_spec))
        operands.append(w_stack)
    else:
        in_specs.append(pl.BlockSpec((1, _D, _D), lambda i: (0, 0, 0)))
        operands.append(jnp.zeros((1, _D, _D), jnp.float32))
    in_specs.append(pl.BlockSpec((1, 1), _const_spec))
    operands.append(alpha)

    out_shape = [jax.ShapeDtypeStruct((_N, _D), jnp.float32)]
    out_specs = [pl.BlockSpec((_BM, _D), _row_spec)]
    if emit_ego:
        out_shape.append(jax.ShapeDtypeStruct((_N, _D), jnp.bfloat16))
        out_specs.append(pl.BlockSpec((_BM, _D), _row_spec))
    if cast_a:
        out_shape.append(jax.ShapeDtypeStruct((_N, _N), jnp.bfloat16))
        out_specs.append(pl.BlockSpec((_BM, _N), _row_spec))

    def body(a_ref, x_ref, w_ref, alpha_ref, *outs):
        emb_ref = outs[0]
        ego_ref = outs[1] if emit_ego else None
        abf_ref = outs[-1] if cast_a else None
        _layer_body(a_ref, x_ref, w_ref, alpha_ref, emb_ref, ego_ref, abf_ref,
                    cast_a=cast_a, emit_ego=emit_ego)

    return pl.pallas_call(
        body,
        grid=(_NB,),
        in_specs=in_specs,
        out_specs=out_specs,
        out_shape=out_shape,
        compiler_params=pltpu.CompilerParams(
            dimension_semantics=("parallel",)),
    )(*operands)


def kernel(user_social_feat, user_feat, item_feat, raitng_adj,
           user1_w, user2_w, item_w, user_w1, item_w1, user_w2, item_w2,
           prelu_a):
    ego0 = pl.pallas_call(
        _proj_body,
        out_shape=jax.ShapeDtypeStruct((_N, _D), jnp.bfloat16),
    )(user_feat, user1_w, user_social_feat, user2_w, item_feat, item_w)

    alpha = jnp.reshape(prelu_a, (1, 1))
    w1 = jnp.stack([user_w1, item_w1])
    w2 = jnp.stack([user_w2, item_w2])

    emb0, ego1, a_bf = _layer_call(raitng_adj, ego0, w1, alpha,
                                   cast_a=True, emit_ego=True)
    emb1, ego2 = _layer_call(a_bf, ego1, w2, alpha,
                             cast_a=False, emit_ego=True)
    (emb2,) = _layer_call(a_bf, ego2, None, alpha,
                          cast_a=False, emit_ego=False)

    user_embedding = jnp.concatenate(
        [emb0[:_USER], emb1[:_USER], emb2[:_USER]], axis=1)
    item_embedding = jnp.concatenate(
        [emb0[_USER:], emb1[_USER:], emb2[_USER:]], axis=1)
    return (user_embedding, item_embedding)


# speedup vs baseline: 1.1160x; 1.0324x over previous
import jax
import jax.numpy as jnp
from jax.experimental import pallas as pl
from jax.experimental.pallas import tpu as pltpu

_USER = 4096
_N = 8192
_D = 16
_BM = 512
_NB = _N // _BM
_NBU = _USER // _BM


def _proj_body(uf_ref, u1w_ref, usf_ref, u2w_ref, itf_ref, iw_ref, out_ref):
    ue1 = jnp.dot(uf_ref[...], u1w_ref[...], preferred_element_type=jnp.float32)
    ue2 = jnp.dot(usf_ref[...], u2w_ref[...], preferred_element_type=jnp.float32)
    ie = jnp.dot(itf_ref[...], iw_ref[...], preferred_element_type=jnp.float32)
    ue = jnp.concatenate([ue1, ue2], axis=1)
    out_ref[...] = jnp.concatenate([ue, ie], axis=0).astype(jnp.bfloat16)


def _pass1_body(a_ref, x_ref, w_ref, alpha_ref, emb_ref, ego_ref, abf_ref):
    a = a_ref[...].astype(jnp.bfloat16)
    abf_ref[...] = a
    acc = jnp.dot(a, x_ref[...], preferred_element_type=jnp.float32)
    alpha = alpha_ref[0, 0]
    emb = jnp.where(acc >= 0, acc, alpha * acc)
    emb_ref[...] = emb
    ego = jnp.dot(emb, w_ref[0], preferred_element_type=jnp.float32)
    ego_ref[...] = ego.astype(jnp.bfloat16)


def _p23_body(a_ref, x1_ref, w_ref, alpha_ref, out_ref, x2_s):
    p = pl.program_id(0)
    i = pl.program_id(1)
    x = jnp.where(p == 0, x1_ref[...], x2_s[...])
    acc = jnp.dot(a_ref[...], x, preferred_element_type=jnp.float32)
    alpha = alpha_ref[0, 0]
    emb = jnp.where(acc >= 0, acc, alpha * acc)
    out_ref[...] = emb[None]

    @pl.when(p == 0)
    def _():
        ego = jnp.dot(emb, w_ref[0], preferred_element_type=jnp.float32)
        x2_s[pl.ds(i * _BM, _BM), :] = ego.astype(jnp.bfloat16)


def kernel(user_social_feat, user_feat, item_feat, raitng_adj,
           user1_w, user2_w, item_w, user_w1, item_w1, user_w2, item_w2,
           prelu_a):
    ego0 = pl.pallas_call(
        _proj_body,
        out_shape=jax.ShapeDtypeStruct((_N, _D), jnp.bfloat16),
    )(user_feat, user1_w, user_social_feat, user2_w, item_feat, item_w)

    alpha = jnp.reshape(prelu_a, (1, 1))
    w1 = jnp.stack([user_w1, item_w1])
    w2 = jnp.stack([user_w2, item_w2])

    emb0, ego1, a_bf = pl.pallas_call(
        _pass1_body,
        grid=(_NB,),
        in_specs=[
            pl.BlockSpec((_BM, _N), lambda i: (i, 0)),
            pl.BlockSpec((_N, _D), lambda i: (0, 0)),
            pl.BlockSpec((1, _D, _D), lambda i: (jnp.where(i < _NBU, 0, 1),
                                                 0, 0)),
            pl.BlockSpec((1, 1), lambda i: (0, 0)),
        ],
        out_specs=[
            pl.BlockSpec((_BM, _D), lambda i: (i, 0)),
            pl.BlockSpec((_BM, _D), lambda i: (i, 0)),
            pl.BlockSpec((_BM, _N), lambda i: (i, 0)),
        ],
        out_shape=[
            jax.ShapeDtypeStruct((_N, _D), jnp.float32),
            jax.ShapeDtypeStruct((_N, _D), jnp.bfloat16),
            jax.ShapeDtypeStruct((_N, _N), jnp.bfloat16),
        ],
        compiler_params=pltpu.CompilerParams(
            dimension_semantics=("arbitrary",),
            vmem_limit_bytes=100 * 1024 * 1024),
    )(raitng_adj, ego0, w1, alpha)

    emb12 = pl.pallas_call(
        _p23_body,
        grid=(2, _NB),
        in_specs=[
            pl.BlockSpec((_BM, _N), lambda p, i: (i, 0)),
            pl.BlockSpec((_N, _D), lambda p, i: (0, 0)),
            pl.BlockSpec((1, _D, _D), lambda p, i: (jnp.where(i < _NBU, 0, 1),
                                                    0, 0)),
            pl.BlockSpec((1, 1), lambda p, i: (0, 0)),
        ],
        out_specs=pl.BlockSpec((1, _BM, _D), lambda p, i: (p, i, 0)),
        out_shape=jax.ShapeDtypeStruct((2, _N, _D), jnp.float32),
        scratch_shapes=[pltpu.VMEM((_N, _D), jnp.bfloat16)],
        compiler_params=pltpu.CompilerParams(
            dimension_semantics=("arbitrary", "arbitrary"),
            vmem_limit_bytes=100 * 1024 * 1024),
    )(a_bf, ego1, w2, alpha)

    emb1 = emb12[0]
    emb2 = emb12[1]
    user_embedding = jnp.concatenate(
        [emb0[:_USER], emb1[:_USER], emb2[:_USER]], axis=1)
    item_embedding = jnp.concatenate(
        [emb0[_USER:], emb1[_USER:], emb2[_USER:]], axis=1)
    return (user_embedding, item_embedding)


# speedup vs baseline: 1.2240x; 1.0968x over previous
import jax
import jax.numpy as jnp
from jax import lax
from jax.experimental import pallas as pl
from jax.experimental.pallas import tpu as pltpu

_USER = 4096
_N = 8192
_D = 16
_BM = 256
_NB = _N // _BM
_NBU = _USER // _BM
_DF = 3
_DB = 4


def _body(a_hbm, uf_ref, u1w_ref, usf_ref, u2w_ref, itf_ref, iw_ref,
          w1u_ref, w1i_ref, w2u_ref, w2i_ref, alpha_ref,
          user_ref, item_ref, abf_hbm,
          fbuf, bbuf, xs, in_sem, wsem, rsem):
    alpha = alpha_ref[0, 0]

    ue1 = jnp.dot(uf_ref[...], u1w_ref[...], preferred_element_type=jnp.float32)
    ue2 = jnp.dot(usf_ref[...], u2w_ref[...], preferred_element_type=jnp.float32)
    ie = jnp.dot(itf_ref[...], iw_ref[...], preferred_element_type=jnp.float32)
    xs[0, :_USER, :] = jnp.concatenate([ue1, ue2], axis=1).astype(jnp.bfloat16)
    xs[0, _USER:, :] = ie.astype(jnp.bfloat16)

    def in_copy(i, b):
        return pltpu.make_async_copy(
            a_hbm.at[pl.ds(i * _BM, _BM), :], fbuf.at[b], in_sem.at[b])

    def out_copy(i, s):
        return pltpu.make_async_copy(
            bbuf.at[s], abf_hbm.at[pl.ds(i * _BM, _BM), :], wsem.at[s])

    def rd_copy(i, s):
        return pltpu.make_async_copy(
            abf_hbm.at[pl.ds(i * _BM, _BM), :], bbuf.at[s], rsem.at[s])

    def store_emb(i, p, emb):
        cols = slice(p * _D, (p + 1) * _D)

        @pl.when(i < _NBU)
        def _():
            user_ref[pl.ds(i * _BM, _BM), cols] = emb

        @pl.when(i >= _NBU)
        def _():
            item_ref[pl.ds((i - _NBU) * _BM, _BM), cols] = emb

    for b in range(_DF - 1):
        in_copy(b, b).start()
    x0 = xs[0][...]
    w1u = w1u_ref[...]
    w1i = w1i_ref[...]

    def step1(i, carry):
        b = lax.rem(i, _DF)
        s = lax.rem(i, _DB)

        @pl.when(i + _DF - 1 < _NB)
        def _():
            in_copy(i + _DF - 1, lax.rem(i + _DF - 1, _DF)).start()

        @pl.when(i >= _DB)
        def _():
            out_copy(i - _DB, s).wait()

        in_copy(i, b).wait()
        a = fbuf[b][...].astype(jnp.bfloat16)
        bbuf[s] = a
        out_copy(i, s).start()
        acc = jnp.dot(a, x0, preferred_element_type=jnp.float32)
        emb = jnp.where(acc >= 0, acc, alpha * acc)
        store_emb(i, 0, emb)
        w = jnp.where(i < _NBU, w1u, w1i)
        xs[1, pl.ds(i * _BM, _BM), :] = jnp.dot(
            emb, w, preferred_element_type=jnp.float32).astype(jnp.bfloat16)
        return carry

    lax.fori_loop(0, _NB, step1, 0)
    for k in range(_DB):
        out_copy(_NB - _DB + k, k).wait()

    def stream_pass(p, wu, wi):
        for k in range(_DB - 1):
            rd_copy(k, k).start()
        x = xs[p][...]

        def step(i, carry):
            s = lax.rem(i, _DB)

            @pl.when(i + _DB - 1 < _NB)
            def _():
                rd_copy(i + _DB - 1, lax.rem(i + _DB - 1, _DB)).start()

            rd_copy(i, s).wait()
            acc = jnp.dot(bbuf[s][...], x, preferred_element_type=jnp.float32)
            emb = jnp.where(acc >= 0, acc, alpha * acc)
            store_emb(i, p, emb)
            if wu is not None:
                w = jnp.where(i < _NBU, wu, wi)
                xs[p + 1, pl.ds(i * _BM, _BM), :] = jnp.dot(
                    emb, w, preferred_element_type=jnp.float32
                ).astype(jnp.bfloat16)
            return carry

        lax.fori_loop(0, _NB, step, 0)

    stream_pass(1, w2u_ref[...], w2i_ref[...])
    stream_pass(2, None, None)


def kernel(user_social_feat, user_feat, item_feat, raitng_adj,
           user1_w, user2_w, item_w, user_w1, item_w1, user_w2, item_w2,
           prelu_a):
    alpha = jnp.reshape(prelu_a, (1, 1))
    user_emb, item_emb, _ = pl.pallas_call(
        _body,
        in_specs=[
            pl.BlockSpec(memory_space=pltpu.MemorySpace.HBM),
            pl.BlockSpec((_USER, 128), lambda: (0, 0)),
            pl.BlockSpec((128, _D // 2), lambda: (0, 0)),
            pl.BlockSpec((_USER, 128), lambda: (0, 0)),
            pl.BlockSpec((128, _D // 2), lambda: (0, 0)),
            pl.BlockSpec((_USER, 128), lambda: (0, 0)),
            pl.BlockSpec((128, _D), lambda: (0, 0)),
            pl.BlockSpec((_D, _D), lambda: (0, 0)),
            pl.BlockSpec((_D, _D), lambda: (0, 0)),
            pl.BlockSpec((_D, _D), lambda: (0, 0)),
            pl.BlockSpec((_D, _D), lambda: (0, 0)),
            pl.BlockSpec((1, 1), lambda: (0, 0)),
        ],
        out_specs=[
            pl.BlockSpec((_USER, 3 * _D), lambda: (0, 0)),
            pl.BlockSpec((_USER, 3 * _D), lambda: (0, 0)),
            pl.BlockSpec(memory_space=pltpu.MemorySpace.HBM),
        ],
        out_shape=[
            jax.ShapeDtypeStruct((_USER, 3 * _D), jnp.float32),
            jax.ShapeDtypeStruct((_USER, 3 * _D), jnp.float32),
            jax.ShapeDtypeStruct((_N, _N), jnp.bfloat16),
        ],
        scratch_shapes=[
            pltpu.VMEM((_DF, _BM, _N), jnp.float32),
            pltpu.VMEM((_DB, _BM, _N), jnp.bfloat16),
            pltpu.VMEM((3, _N, _D), jnp.bfloat16),
            pltpu.SemaphoreType.DMA((_DF,)),
            pltpu.SemaphoreType.DMA((_DB,)),
            pltpu.SemaphoreType.DMA((_DB,)),
        ],
        compiler_params=pltpu.CompilerParams(
            vmem_limit_bytes=100 * 1024 * 1024),
    )(raitng_adj, user_feat, user1_w, user_social_feat, user2_w, item_feat,
      item_w, user_w1, item_w1, user_w2, item_w2, alpha)
    return (user_emb, item_emb)
